# Initial kernel scaffold; baseline (speedup 1.0000x reference)
import jax
import jax.numpy as jnp
from jax.experimental import pallas as pl


def kernel(params, inds, wgts, freqs):
    raise NotImplementedError("write your pallas kernel here")


# speedup vs baseline: 1.4445x; 1.4445x over previous
import functools

import jax
import jax.numpy as jnp
from jax import lax
from jax.experimental import pallas as pl
from jax.experimental.pallas import tpu as pltpu
from jax.experimental.pallas import tpu_sc as plsc

NPIX = 196608
NFREQS = 128
NPTS = 65536

NW = 32
PTS_PER_W = NPTS // NW
PTS_PER_CHUNK = 32
ROWS_PER_CHUNK = PTS_PER_CHUNK * 4
CHUNKS = PTS_PER_W // PTS_PER_CHUNK
LANES = 16
SLICES = NFREQS // LANES


def _sc_gather(table, idx3, wgt3):
    mesh = plsc.VectorSubcoreMesh(core_axis_name="c", subcore_axis_name="s")

    @functools.partial(
        pl.kernel,
        out_type=jax.ShapeDtypeStruct((NPTS, NFREQS), jnp.float32),
        mesh=mesh,
        scratch_types=[
            pltpu.VMEM((CHUNKS, ROWS_PER_CHUNK), jnp.int32),
            pltpu.VMEM((CHUNKS, ROWS_PER_CHUNK), jnp.float32),
            pltpu.VMEM((ROWS_PER_CHUNK, NFREQS), jnp.float32),
            pltpu.VMEM((PTS_PER_CHUNK, NFREQS), jnp.float32),
            pltpu.SemaphoreType.DMA,
        ],
    )
    def k(table_hbm, idx_hbm, wgt_hbm, out_hbm, idx_v, wgt_v, buf, outb, sem):
        wid = lax.axis_index("s") * 2 + lax.axis_index("c")
        base = wid * PTS_PER_W
        pltpu.sync_copy(idx_hbm.at[wid], idx_v)
        pltpu.sync_copy(wgt_hbm.at[wid], wgt_v)

        def chunk_body(g, carry):
            pltpu.async_copy(table_hbm.at[idx_v.at[g]], buf, sem).wait()

            def quad_body(q, c):
                wv = wgt_v[g, pl.ds(q * LANES, LANES)]
                for pp in range(4):
                    p = q * 4 + pp
                    w = [
                        jnp.full((LANES,), wv[4 * pp + j], dtype=jnp.float32)
                        for j in range(4)
                    ]
                    for s in range(SLICES):
                        acc = w[0] * buf[4 * p + 0, pl.ds(s * LANES, LANES)]
                        for j in range(1, 4):
                            acc = acc + w[j] * buf[4 * p + j, pl.ds(s * LANES, LANES)]
                        outb[p, pl.ds(s * LANES, LANES)] = acc
                return c

            lax.fori_loop(0, PTS_PER_CHUNK // 4, quad_body, 0, unroll=False)
            pltpu.sync_copy(
                outb, out_hbm.at[pl.ds(base + g * PTS_PER_CHUNK, PTS_PER_CHUNK)]
            )
            return carry

        lax.fori_loop(0, CHUNKS, chunk_body, 0, unroll=False)

    return k(table, idx3, wgt3)


def kernel(params, inds, wgts, freqs):
    table = params.reshape(NFREQS, NPIX).T
    idx3 = inds.astype(jnp.int32).reshape(NW, CHUNKS, ROWS_PER_CHUNK)
    wgt3 = wgts.astype(jnp.float32).reshape(NW, CHUNKS, ROWS_PER_CHUNK)
    out = _sc_gather(table, idx3, wgt3)
    return out.T.reshape(1, 1, 1, NFREQS, NPTS)


# speedup vs baseline: 1.8928x; 1.3103x over previous
import functools

import jax
import jax.numpy as jnp
from jax import lax
from jax.experimental import pallas as pl
from jax.experimental.pallas import tpu as pltpu
from jax.experimental.pallas import tpu_sc as plsc

NPIX = 196608
NFREQS = 128
NPTS = 65536

NW = 32
PTS_PER_W = NPTS // NW
PTS_PER_CHUNK = 32
ROWS_PER_CHUNK = PTS_PER_CHUNK * 4
CHUNKS = PTS_PER_W // PTS_PER_CHUNK
LANES = 16
SLICES = NFREQS // LANES


def _sc_gather(table, idx3, wgt3):
    mesh = plsc.VectorSubcoreMesh(core_axis_name="c", subcore_axis_name="s")

    @functools.partial(
        pl.kernel,
        out_type=jax.ShapeDtypeStruct((NPTS, NFREQS), jnp.float32),
        mesh=mesh,
        scratch_types=[
            pltpu.VMEM((CHUNKS, ROWS_PER_CHUNK), jnp.int32),
            pltpu.VMEM((CHUNKS, ROWS_PER_CHUNK), jnp.float32),
            pltpu.VMEM((2, ROWS_PER_CHUNK, NFREQS), jnp.float32),
            pltpu.VMEM((2, PTS_PER_CHUNK, NFREQS), jnp.float32),
            pltpu.SemaphoreType.DMA,
            pltpu.SemaphoreType.DMA,
            pltpu.SemaphoreType.DMA,
            pltpu.SemaphoreType.DMA,
        ],
    )
    def k(table_hbm, idx_hbm, wgt_hbm, out_hbm, idx_v, wgt_v, buf, outb,
          gsem0, gsem1, osem0, osem1):
        gsems = (gsem0, gsem1)
        osems = (osem0, osem1)
        wid = lax.axis_index("s") * 2 + lax.axis_index("c")
        base = wid * PTS_PER_W
        pltpu.sync_copy(idx_hbm.at[wid], idx_v)
        pltpu.sync_copy(wgt_hbm.at[wid], wgt_v)

        pltpu.async_copy(table_hbm.at[idx_v.at[0]], buf.at[0], gsems[0])
        pltpu.async_copy(table_hbm.at[idx_v.at[1]], buf.at[1], gsems[1])

        def pair_body(h, carry):
            for b in range(2):
                g = 2 * h + b
                pltpu.make_async_copy(
                    table_hbm.at[idx_v.at[g]], buf.at[b], gsems[b]
                ).wait()
                @pl.when(h >= 1)
                def _():
                    pltpu.make_async_copy(
                        outb.at[b],
                        out_hbm.at[pl.ds(base + (g - 2) * PTS_PER_CHUNK,
                                         PTS_PER_CHUNK)],
                        osems[b],
                    ).wait()

                def quad_body(q, c):
                    wv = wgt_v[g, pl.ds(q * LANES, LANES)]
                    for pp in range(4):
                        p = q * 4 + pp
                        w = [
                            jnp.full((LANES,), wv[4 * pp + j], dtype=jnp.float32)
                            for j in range(4)
                        ]
                        for s in range(SLICES):
                            acc = w[0] * buf[b, 4 * p + 0, pl.ds(s * LANES, LANES)]
                            for j in range(1, 4):
                                acc = acc + w[j] * buf[b, 4 * p + j,
                                                       pl.ds(s * LANES, LANES)]
                            outb[b, p, pl.ds(s * LANES, LANES)] = acc
                    return c

                lax.fori_loop(0, PTS_PER_CHUNK // 4, quad_body, 0, unroll=False)

                @pl.when(g + 2 < CHUNKS)
                def _():
                    pltpu.async_copy(
                        table_hbm.at[idx_v.at[g + 2]], buf.at[b], gsems[b]
                    )

                pltpu.async_copy(
                    outb.at[b],
                    out_hbm.at[pl.ds(base + g * PTS_PER_CHUNK, PTS_PER_CHUNK)],
                    osems[b],
                )
            return carry

        lax.fori_loop(0, CHUNKS // 2, pair_body, 0, unroll=False)

        for b in range(2):
            pltpu.make_async_copy(
                outb.at[b],
                out_hbm.at[pl.ds(base + (CHUNKS - 2 + b) * PTS_PER_CHUNK,
                                 PTS_PER_CHUNK)],
                osems[b],
            ).wait()

    return k(table, idx3, wgt3)


def kernel(params, inds, wgts, freqs):
    table = params.reshape(NFREQS, NPIX).T
    idx3 = inds.astype(jnp.int32).reshape(NW, CHUNKS, ROWS_PER_CHUNK)
    wgt3 = wgts.astype(jnp.float32).reshape(NW, CHUNKS, ROWS_PER_CHUNK)
    out = _sc_gather(table, idx3, wgt3)
    return out.T.reshape(1, 1, 1, NFREQS, NPTS)
